# baseline (speedup 1.0000x reference)
import functools
import math

import jax
import jax.numpy as jnp
from jax import lax
from jax.experimental import pallas as pl
from jax.experimental.pallas import tpu as pltpu
from jax.experimental.pallas import tpu_sc as plsc

D_MODEL = 64
SCALE = math.sqrt(D_MODEL)

_info = plsc.get_sparse_core_info()
NC, NS, L = _info.num_cores, _info.num_subcores, _info.num_lanes
NW = NC * NS

VOC = 1000000
NB2 = VOC // 256
PER = NB2 // NW
EXTRA = NB2 - PER * NW

TOK = 128
NBUF = 4
LEAD = 2
SBUF = 2

_mesh = plsc.VectorSubcoreMesh(core_axis_name="c", subcore_axis_name="s")


@functools.partial(
    pl.kernel,
    out_type=jax.ShapeDtypeStruct((VOC // 2, 128), jnp.float32),
    mesh=_mesh,
    scratch_types=[
        [pltpu.VMEM((64, 256), jnp.float32) for _ in range(2)],
        [pltpu.VMEM((128, 128), jnp.float32) for _ in range(2)],
        pltpu.VMEM((64, 64), jnp.float32),
        pltpu.VMEM((32, 128), jnp.float32),
        [pltpu.SemaphoreType.DMA for _ in range(2)],
        [pltpu.SemaphoreType.DMA for _ in range(2)],
        pltpu.SemaphoreType.DMA,
    ],
    compiler_params=pltpu.CompilerParams(
        use_tc_tiling_on_sc=True, needs_layout_passes=False
    ),
)
def _k1(lutT_hbm, scr_hbm, cb, ob, tcb, tob, rsem, wsem, tsem):
    wid = lax.axis_index("s") * NC + lax.axis_index("c")
    c0 = wid * PER

    def rstart(c, s):
        pltpu.async_copy(lutT_hbm.at[:, pl.ds(c * 256, 256)], cb[s], rsem[s])

    def rwait(c, s):
        pltpu.make_async_copy(
            lutT_hbm.at[:, pl.ds(c * 256, 256)], cb[s], rsem[s]
        ).wait()

    def wstart(c, s):
        pltpu.async_copy(ob[s], scr_hbm.at[pl.ds(c * 128, 128)], wsem[s])

    def wwait(c, s):
        pltpu.make_async_copy(
            ob[s], scr_hbm.at[pl.ds(c * 128, 128)], wsem[s]
        ).wait()

    def transpose2(s):
        @plsc.parallel_loop(0, 128, unroll=4)
        def _(p):
            for h in range(2):
                col = jnp.full((L,), 0, jnp.int32) + (2 * p + h)
                for q0 in range(0, 64, L):
                    rows = jax.lax.iota(jnp.int32, L) + q0
                    vals = plsc.load_gather(cb[s], [rows, col])
                    ob[s][p, pl.ds(h * 64 + q0, L)] = vals * SCALE

    rstart(c0, 0)

    def group(g, carry):
        for b in range(2):
            i = g * 2 + b
            c = c0 + i

            @pl.when(i + 1 < PER)
            def _():
                rstart(c + 1, 1 - b)

            rwait(c, b)

            @pl.when(i >= 2)
            def _():
                wwait(c - 2, b)

            transpose2(b)
            wstart(c, b)
        return carry

    lax.fori_loop(0, PER // 2, group, 0)
    wwait(c0 + PER - 2, 0)
    wwait(c0 + PER - 1, 1)

    @pl.when(wid < EXTRA)
    def _():
        c = PER * NW + wid
        rstart(c, 0)
        rwait(c, 0)
        transpose2(0)
        wstart(c, 0)
        wwait(c, 0)

    @pl.when(wid == EXTRA)
    def _():
        pltpu.async_copy(
            lutT_hbm.at[:, pl.ds(VOC - 64, 64)], tcb, tsem
        ).wait()

        @plsc.parallel_loop(0, 32, unroll=2)
        def _(p):
            for h in range(2):
                col = jnp.full((L,), 0, jnp.int32) + (2 * p + h)
                for q0 in range(0, 64, L):
                    rows = jax.lax.iota(jnp.int32, L) + q0
                    vals = plsc.load_gather(tcb, [rows, col])
                    tob[p, pl.ds(h * 64 + q0, L)] = vals * SCALE

        pltpu.async_copy(
            tob, scr_hbm.at[pl.ds((VOC - 64) // 2, 32)], tsem
        ).wait()


def _make_k2(NI, NJ):
    assert NI == NW * TOK

    @functools.partial(
        pl.kernel,
        out_type=jax.ShapeDtypeStruct((NJ, 8, NI // 128, 8, 128), jnp.float32),
        mesh=_mesh,
        scratch_types=[
            pltpu.VMEM((NJ, TOK), jnp.int32),
            [pltpu.VMEM((TOK, D_MODEL), jnp.float32) for _ in range(NBUF)],
            [pltpu.VMEM((1, 8, 1, 8, 128), jnp.float32) for _ in range(SBUF)],
            [pltpu.VMEM((TOK,), jnp.int32) for _ in range(NBUF)],
            pltpu.SemaphoreType.DMA,
            [pltpu.SemaphoreType.DMA for _ in range(NBUF)],
            [pltpu.SemaphoreType.DMA for _ in range(SBUF)],
        ],
        compiler_params=pltpu.CompilerParams(
            use_tc_tiling_on_sc=False, needs_layout_passes=False
        ),
    )
    def k2(lut_hbm, xt_hbm, out_hbm, xblk, gbuf, tbuf, ibuf, xsem, gsem, ssem):
        wid = lax.axis_index("s") * NC + lax.axis_index("c")
        i0 = wid * TOK
        pltpu.async_copy(xt_hbm.at[:, pl.ds(i0, TOK)], xblk, xsem).wait()

        def prep_idx(j, slot):
            for t0 in range(TOK // L):
                sl = pl.ds(t0 * L, L)
                ibuf[slot][sl] = xblk[j, sl]

        def gather_start(slot):
            pltpu.async_copy(lut_hbm.at[ibuf[slot]], gbuf[slot], gsem[slot])

        def gather_wait(slot):
            pltpu.make_async_copy(
                lut_hbm.at[ibuf[slot]], gbuf[slot], gsem[slot]
            ).wait()

        def scatter_start(j, slot):
            pltpu.async_copy(
                tbuf[slot],
                out_hbm.at[pl.ds(j, 1), :, pl.ds(wid, 1)],
                ssem[slot],
            )

        def scatter_wait(j, slot):
            pltpu.make_async_copy(
                tbuf[slot],
                out_hbm.at[pl.ds(j, 1), :, pl.ds(wid, 1)],
                ssem[slot],
            ).wait()

        def transpose_block(gslot, tslot):
            @plsc.parallel_loop(0, TOK // L, unroll=8)
            def _(t0):
                rows = jax.lax.iota(jnp.int32, L) + t0 * L
                for kf in range(D_MODEL):
                    col = jnp.full((L,), kf, jnp.int32)
                    vals = plsc.load_gather(gbuf[gslot], [rows, col])
                    tbuf[tslot][0, kf // 8, 0, kf % 8, pl.ds(t0 * L, L)] = vals

        for b in range(LEAD):
            prep_idx(b, b)
            gather_start(b)

        def group_body(grp, carry):
            for b in range(NBUF):
                j = grp * NBUF + b
                gather_wait(b)
                ts = b % SBUF

                @pl.when(j >= SBUF)
                def _():
                    scatter_wait(j - SBUF, ts)

                transpose_block(b, ts)
                scatter_start(j, ts)

                h = j + LEAD
                sb = (b + LEAD) % NBUF

                @pl.when(h < NJ)
                def _():
                    prep_idx(h, sb)
                    gather_start(sb)

            return carry

        lax.fori_loop(0, NJ // NBUF, group_body, 0)

        for j in range(NJ - SBUF, NJ):
            scatter_wait(j, j % SBUF)

    return k2


def kernel(x, lut):
    NI, NJ = x.shape
    scr = _k1(lut.T)
    lutr = scr.reshape(VOC, D_MODEL)
    xt = x.T.astype(jnp.int32)
    out5 = _make_k2(NI, NJ)(lutr, xt)
    return out5.transpose(2, 4, 0, 1, 3).reshape(NI, NJ, D_MODEL)


# speedup vs baseline: 1.0374x; 1.0374x over previous
import functools
import math

import jax
import jax.numpy as jnp
from jax import lax
from jax.experimental import pallas as pl
from jax.experimental.pallas import tpu as pltpu
from jax.experimental.pallas import tpu_sc as plsc

D_MODEL = 64
SCALE = math.sqrt(D_MODEL)

_info = plsc.get_sparse_core_info()
NC, NS, L = _info.num_cores, _info.num_subcores, _info.num_lanes
NW = NC * NS

VOC = 1000000
NB2 = VOC // 256
PER = NB2 // NW
EXTRA = NB2 - PER * NW

TOK = 128
NBUF = 4
LEAD = 2
SBUF = 2

_mesh = plsc.VectorSubcoreMesh(core_axis_name="c", subcore_axis_name="s")


@functools.partial(
    pl.kernel,
    out_type=jax.ShapeDtypeStruct((VOC // 2, 128), jnp.float32),
    mesh=_mesh,
    scratch_types=[
        [pltpu.VMEM((64, 256), jnp.float32) for _ in range(2)],
        [pltpu.VMEM((128, 128), jnp.float32) for _ in range(2)],
        pltpu.VMEM((64, 64), jnp.float32),
        pltpu.VMEM((32, 128), jnp.float32),
        [pltpu.SemaphoreType.DMA for _ in range(2)],
        [pltpu.SemaphoreType.DMA for _ in range(2)],
        pltpu.SemaphoreType.DMA,
    ],
    compiler_params=pltpu.CompilerParams(
        use_tc_tiling_on_sc=True, needs_layout_passes=False
    ),
)
def _k1(lutT_hbm, scr_hbm, cb, ob, tcb, tob, rsem, wsem, tsem):
    wid = lax.axis_index("s") * NC + lax.axis_index("c")
    c0 = wid * PER

    def rstart(c, s):
        pltpu.async_copy(lutT_hbm.at[:, pl.ds(c * 256, 256)], cb[s], rsem[s])

    def rwait(c, s):
        pltpu.make_async_copy(
            lutT_hbm.at[:, pl.ds(c * 256, 256)], cb[s], rsem[s]
        ).wait()

    def wstart(c, s):
        pltpu.async_copy(ob[s], scr_hbm.at[pl.ds(c * 128, 128)], wsem[s])

    def wwait(c, s):
        pltpu.make_async_copy(
            ob[s], scr_hbm.at[pl.ds(c * 128, 128)], wsem[s]
        ).wait()

    def transpose2(s):
        @plsc.parallel_loop(0, 128, unroll=2)
        def _(p):
            loads = []
            for h in range(2):
                col = jnp.full((L,), 0, jnp.int32) + (2 * p + h)
                for q0 in range(0, 64, L):
                    rows = jax.lax.iota(jnp.int32, L) + q0
                    loads.append((h, q0, plsc.load_gather(cb[s], [rows, col])))
            for h, q0, vals in loads:
                ob[s][p, pl.ds(h * 64 + q0, L)] = vals * SCALE

    rstart(c0, 0)

    def group(g, carry):
        for b in range(2):
            i = g * 2 + b
            c = c0 + i

            @pl.when(i + 1 < PER)
            def _():
                rstart(c + 1, 1 - b)

            rwait(c, b)

            @pl.when(i >= 2)
            def _():
                wwait(c - 2, b)

            transpose2(b)
            wstart(c, b)
        return carry

    lax.fori_loop(0, PER // 2, group, 0)
    wwait(c0 + PER - 2, 0)
    wwait(c0 + PER - 1, 1)

    @pl.when(wid < EXTRA)
    def _():
        c = PER * NW + wid
        rstart(c, 0)
        rwait(c, 0)
        transpose2(0)
        wstart(c, 0)
        wwait(c, 0)

    @pl.when(wid == EXTRA)
    def _():
        pltpu.async_copy(
            lutT_hbm.at[:, pl.ds(VOC - 64, 64)], tcb, tsem
        ).wait()

        @plsc.parallel_loop(0, 32, unroll=2)
        def _(p):
            for h in range(2):
                col = jnp.full((L,), 0, jnp.int32) + (2 * p + h)
                for q0 in range(0, 64, L):
                    rows = jax.lax.iota(jnp.int32, L) + q0
                    vals = plsc.load_gather(tcb, [rows, col])
                    tob[p, pl.ds(h * 64 + q0, L)] = vals * SCALE

        pltpu.async_copy(
            tob, scr_hbm.at[pl.ds((VOC - 64) // 2, 32)], tsem
        ).wait()


def _make_k2(NI, NJ):
    assert NI == NW * TOK

    @functools.partial(
        pl.kernel,
        out_type=jax.ShapeDtypeStruct((NJ, 8, NI // 128, 8, 128), jnp.float32),
        mesh=_mesh,
        scratch_types=[
            pltpu.VMEM((NJ, TOK), jnp.int32),
            [pltpu.VMEM((TOK, D_MODEL), jnp.float32) for _ in range(NBUF)],
            [pltpu.VMEM((1, 8, 1, 8, 128), jnp.float32) for _ in range(SBUF)],
            [pltpu.VMEM((TOK,), jnp.int32) for _ in range(NBUF)],
            pltpu.SemaphoreType.DMA,
            [pltpu.SemaphoreType.DMA for _ in range(NBUF)],
            [pltpu.SemaphoreType.DMA for _ in range(SBUF)],
        ],
        compiler_params=pltpu.CompilerParams(
            use_tc_tiling_on_sc=False, needs_layout_passes=False
        ),
    )
    def k2(lut_hbm, xt_hbm, out_hbm, xblk, gbuf, tbuf, ibuf, xsem, gsem, ssem):
        wid = lax.axis_index("s") * NC + lax.axis_index("c")
        i0 = wid * TOK
        pltpu.async_copy(xt_hbm.at[:, pl.ds(i0, TOK)], xblk, xsem).wait()

        def prep_idx(j, slot):
            for t0 in range(TOK // L):
                sl = pl.ds(t0 * L, L)
                ibuf[slot][sl] = xblk[j, sl]

        def gather_start(slot):
            pltpu.async_copy(lut_hbm.at[ibuf[slot]], gbuf[slot], gsem[slot])

        def gather_wait(slot):
            pltpu.make_async_copy(
                lut_hbm.at[ibuf[slot]], gbuf[slot], gsem[slot]
            ).wait()

        def scatter_start(j, slot):
            pltpu.async_copy(
                tbuf[slot],
                out_hbm.at[pl.ds(j, 1), :, pl.ds(wid, 1)],
                ssem[slot],
            )

        def scatter_wait(j, slot):
            pltpu.make_async_copy(
                tbuf[slot],
                out_hbm.at[pl.ds(j, 1), :, pl.ds(wid, 1)],
                ssem[slot],
            ).wait()

        def transpose_block(gslot, tslot):
            @plsc.parallel_loop(0, TOK // L, unroll=2)
            def _(t0):
                rows = jax.lax.iota(jnp.int32, L) + t0 * L
                for kg in range(0, D_MODEL, L):
                    vals = [
                        plsc.load_gather(
                            gbuf[gslot], [rows, jnp.full((L,), kf, jnp.int32)]
                        )
                        for kf in range(kg, kg + L)
                    ]
                    for i, kf in enumerate(range(kg, kg + L)):
                        tbuf[tslot][
                            0, kf // 8, 0, kf % 8, pl.ds(t0 * L, L)
                        ] = vals[i]

        for b in range(LEAD):
            prep_idx(b, b)
            gather_start(b)

        def group_body(grp, carry):
            for b in range(NBUF):
                j = grp * NBUF + b
                gather_wait(b)
                ts = b % SBUF

                @pl.when(j >= SBUF)
                def _():
                    scatter_wait(j - SBUF, ts)

                transpose_block(b, ts)
                scatter_start(j, ts)

                h = j + LEAD
                sb = (b + LEAD) % NBUF

                @pl.when(h < NJ)
                def _():
                    prep_idx(h, sb)
                    gather_start(sb)

            return carry

        lax.fori_loop(0, NJ // NBUF, group_body, 0)

        for j in range(NJ - SBUF, NJ):
            scatter_wait(j, j % SBUF)

    return k2


def kernel(x, lut):
    NI, NJ = x.shape
    scr = _k1(lut.T)
    lutr = scr.reshape(VOC, D_MODEL)
    xt = x.T.astype(jnp.int32)
    out5 = _make_k2(NI, NJ)(lutr, xt)
    return out5.transpose(2, 4, 0, 1, 3).reshape(NI, NJ, D_MODEL)


# speedup vs baseline: 1.3777x; 1.3280x over previous
import functools
import math

import jax
import jax.numpy as jnp
from jax import lax
from jax.experimental import pallas as pl
from jax.experimental.pallas import tpu as pltpu
from jax.experimental.pallas import tpu_sc as plsc

D_MODEL = 64
SCALE = math.sqrt(D_MODEL)

_info = plsc.get_sparse_core_info()
NC, NS, L = _info.num_cores, _info.num_subcores, _info.num_lanes
NW = NC * NS

CHUNK = 128
NBUF = 4
LEAD = 2


def _make_kernel(B, D):
    assert B % (NW * CHUNK) == 0
    b_per_w = B // NW
    n_chunks = b_per_w // CHUNK
    assert n_chunks % NBUF == 0 and LEAD < NBUF
    mesh = plsc.VectorSubcoreMesh(core_axis_name="c", subcore_axis_name="s")

    @functools.partial(
        pl.kernel,
        out_type=jax.ShapeDtypeStruct((B, D), jnp.float32),
        mesh=mesh,
        scratch_types=[
            pltpu.VMEM((b_per_w,), jnp.int32),
            [pltpu.VMEM((CHUNK, D), jnp.float32) for _ in range(NBUF)],
            [pltpu.SemaphoreType.DMA for _ in range(NBUF)],
            [pltpu.SemaphoreType.DMA for _ in range(NBUF)],
        ],
        compiler_params=pltpu.CompilerParams(use_tc_tiling_on_sc=False),
    )
    def k(lut_hbm, idx_hbm, out_hbm, idx_v, rows, gsem, ssem):
        wid = lax.axis_index("s") * NC + lax.axis_index("c")
        base = wid * b_per_w
        pltpu.sync_copy(idx_hbm.at[pl.ds(base, b_per_w)], idx_v)

        def gather_start(chunk, slot):
            pltpu.async_copy(
                lut_hbm.at[idx_v.at[pl.ds(chunk * CHUNK, CHUNK)]],
                rows[slot],
                gsem[slot],
            )

        def gather_wait(chunk, slot):
            pltpu.make_async_copy(
                lut_hbm.at[idx_v.at[pl.ds(chunk * CHUNK, CHUNK)]],
                rows[slot],
                gsem[slot],
            ).wait()

        def scatter_start(chunk, slot):
            pltpu.async_copy(
                rows[slot],
                out_hbm.at[pl.ds(base + chunk * CHUNK, CHUNK)],
                ssem[slot],
            )

        def scatter_wait(chunk, slot):
            pltpu.make_async_copy(
                rows[slot],
                out_hbm.at[pl.ds(base + chunk * CHUNK, CHUNK)],
                ssem[slot],
            ).wait()

        for b in range(LEAD):
            gather_start(b, b)

        def group_body(grp, carry):
            for b in range(NBUF):
                g = grp * NBUF + b
                gather_wait(g, b)

                def row_body(j, carry2):
                    for t in range(D // L):
                        sl = pl.ds(t * L, L)
                        rows[b][j, sl] = rows[b][j, sl] * SCALE
                    return carry2

                lax.fori_loop(0, CHUNK, row_body, 0, unroll=2)
                scatter_start(g, b)

                h = g + LEAD
                sb = (b + LEAD) % NBUF

                @pl.when(h < n_chunks)
                def _():
                    @pl.when(h >= NBUF)
                    def _():
                        scatter_wait(h - NBUF, sb)

                    gather_start(h, sb)

            return carry

        lax.fori_loop(0, n_chunks // NBUF, group_body, 0)

        for g in range(n_chunks - NBUF, n_chunks):
            scatter_wait(g, g % NBUF)

    return k


def kernel(x, lut):
    B = x.shape[0] * x.shape[1]
    xflat = x.reshape(B).astype(jnp.int32)
    out = _make_kernel(B, D_MODEL)(lut, xflat)
    return out.reshape(x.shape[0], x.shape[1], D_MODEL)
